# baseline (speedup 1.0000x reference)
import jax
import jax.numpy as jnp
from jax.experimental import pallas as pl


def _copy_body(x_ref, o_ref):
    o_ref[...] = x_ref[...]


def kernel(x, subject_id, gamma_w, beta_w):
    batch, seq, dim = x.shape
    nrows = batch * seq * dim // 128
    x2 = x.reshape(nrows, 128)
    r_blk = 6400
    out2 = pl.pallas_call(
        _copy_body,
        grid=(nrows // r_blk,),
        in_specs=[pl.BlockSpec((r_blk, 128), lambda i: (i, 0))],
        out_specs=pl.BlockSpec((r_blk, 128), lambda i: (i, 0)),
        out_shape=jax.ShapeDtypeStruct((nrows, 128), jnp.float32),
    )(x2)
    return out2.reshape(batch, seq, dim)


# speedup vs baseline: 7.1021x; 7.1021x over previous
import functools

import jax
import jax.numpy as jnp
from jax import lax
from jax.experimental import pallas as pl
from jax.experimental.pallas import tpu as pltpu
from jax.experimental.pallas import tpu_sc as plsc

_DIM = 64
_PACK = 2 * _DIM
_BL = 128


def _sc_worker_count():
    try:
        info = plsc.get_sparse_core_info()
        return info.num_cores, info.num_subcores
    except Exception:
        return 2, 16


def _make_sc_gather(batch, pack):
    nc, ns = _sc_worker_count()
    nw = nc * ns
    b_per_w = batch // nw
    mesh = plsc.VectorSubcoreMesh(core_axis_name="c", subcore_axis_name="s")

    @functools.partial(
        pl.kernel,
        mesh=mesh,
        out_type=jax.ShapeDtypeStruct((batch, pack), jnp.float32),
        scratch_types=[
            pltpu.VMEM((b_per_w,), jnp.int32),
            pltpu.VMEM((b_per_w, pack), jnp.float32),
            pltpu.SemaphoreType.DMA,
        ],
    )
    def sc_gather(table_hbm, idx_hbm, out_hbm, idx_v, rows_v, sem):
        wid = lax.axis_index("s") * nc + lax.axis_index("c")
        base = wid * b_per_w
        pltpu.sync_copy(idx_hbm.at[pl.ds(base, b_per_w)], idx_v)
        pltpu.async_copy(table_hbm.at[idx_v], rows_v, sem).wait()
        pltpu.sync_copy(rows_v, out_hbm.at[pl.ds(base, b_per_w)])

    return sc_gather


def _apply_body(xt_ref, gb_ref, o_ref):
    gb = gb_ref[...]
    eye = (
        lax.broadcasted_iota(jnp.int32, (_BL, _BL), 0)
        == lax.broadcasted_iota(jnp.int32, (_BL, _BL), 1)
    ).astype(jnp.float32)
    gbt = lax.dot_general(
        gb,
        eye,
        (((0,), (0,)), ((), ())),
        preferred_element_type=jnp.float32,
    )
    g = gbt[0:_DIM, :]
    b = gbt[_DIM:_PACK, :]
    o_ref[...] = xt_ref[...] * g[None, :, :] + b[None, :, :]


def kernel(x, subject_id, gamma_w, beta_w):
    batch, seq, dim = x.shape
    idx = subject_id.astype(jnp.int32)
    table = jnp.concatenate([gamma_w, beta_w], axis=1)

    gb = _make_sc_gather(batch, _PACK)(table, idx)

    xt = jnp.transpose(x, (1, 2, 0))

    out_t = pl.pallas_call(
        _apply_body,
        grid=(batch // _BL,),
        in_specs=[
            pl.BlockSpec((seq, dim, _BL), lambda j: (0, 0, j)),
            pl.BlockSpec((_BL, _PACK), lambda j: (j, 0)),
        ],
        out_specs=pl.BlockSpec((seq, dim, _BL), lambda j: (0, 0, j)),
        out_shape=jax.ShapeDtypeStruct((seq, dim, batch), jnp.float32),
    )(xt, gb)
    return jnp.transpose(out_t, (2, 0, 1))


# speedup vs baseline: 7.1148x; 1.0018x over previous
import functools

import jax
import jax.numpy as jnp
from jax import lax
from jax.experimental import pallas as pl
from jax.experimental.pallas import tpu as pltpu
from jax.experimental.pallas import tpu_sc as plsc

_DIM = 64
_PACK = 2 * _DIM
_BL = 128


def _sc_worker_count():
    try:
        info = plsc.get_sparse_core_info()
        return info.num_cores, info.num_subcores
    except Exception:
        return 2, 16


def _make_sc_gather(batch, pack):
    nc, ns = _sc_worker_count()
    nw = nc * ns
    b_per_w = batch // nw
    mesh = plsc.VectorSubcoreMesh(core_axis_name="c", subcore_axis_name="s")

    @functools.partial(
        pl.kernel,
        mesh=mesh,
        out_type=jax.ShapeDtypeStruct((batch, pack), jnp.float32),
        scratch_types=[
            pltpu.VMEM((b_per_w,), jnp.int32),
            pltpu.VMEM((b_per_w, pack), jnp.float32),
            pltpu.SemaphoreType.DMA,
        ],
    )
    def sc_gather(table_hbm, idx_hbm, out_hbm, idx_v, rows_v, sem):
        wid = lax.axis_index("s") * nc + lax.axis_index("c")
        base = wid * b_per_w
        pltpu.sync_copy(idx_hbm.at[pl.ds(base, b_per_w)], idx_v)
        pltpu.async_copy(table_hbm.at[idx_v], rows_v, sem).wait()
        pltpu.sync_copy(rows_v, out_hbm.at[pl.ds(base, b_per_w)])

    return sc_gather


def _apply_body(xt_ref, gb_ref, o_ref):
    gbt = jnp.swapaxes(gb_ref[...], 0, 1)
    g = gbt[0:_DIM, :]
    b = gbt[_DIM:_PACK, :]
    o_ref[...] = xt_ref[...] * g[None, :, :] + b[None, :, :]


def kernel(x, subject_id, gamma_w, beta_w):
    batch, seq, dim = x.shape
    idx = subject_id.astype(jnp.int32)
    table = jnp.concatenate([gamma_w, beta_w], axis=1)

    gb = _make_sc_gather(batch, _PACK)(table, idx)

    xt = jnp.transpose(x, (1, 2, 0))

    out_t = pl.pallas_call(
        _apply_body,
        grid=(batch // _BL,),
        in_specs=[
            pl.BlockSpec((seq, dim, _BL), lambda j: (0, 0, j)),
            pl.BlockSpec((_BL, _PACK), lambda j: (j, 0)),
        ],
        out_specs=pl.BlockSpec((seq, dim, _BL), lambda j: (0, 0, j)),
        out_shape=jax.ShapeDtypeStruct((seq, dim, batch), jnp.float32),
    )(xt, gb)
    return jnp.transpose(out_t, (2, 0, 1))
